# Initial kernel scaffold; baseline (speedup 1.0000x reference)
import jax
import jax.numpy as jnp
from jax.experimental import pallas as pl


def kernel(types, positions, radial_neighbors, c_table):
    raise NotImplementedError("write your pallas kernel here")


# speedup vs baseline: 18.5550x; 18.5550x over previous
import functools

import numpy as np
import jax
import jax.numpy as jnp
from jax import lax
from jax.experimental import pallas as pl
from jax.experimental.pallas import tpu as pltpu
from jax.experimental.pallas import tpu_sc as plsc

_N_ATOMS = 100000
_NN = 16
_N_TYPES = 4
_N_DESC = 16
_K_MAX = 4
_R_C = 5.0

_E = _N_ATOMS * _NN

_NC = 2
_NS = 16
_NW = _NC * _NS
_EPW = _E // _NW
_CH = 10000


def _sc_gather(idx, ptbl):
    mesh = plsc.VectorSubcoreMesh(core_axis_name="c", subcore_axis_name="s")

    @functools.partial(
        pl.kernel,
        mesh=mesh,
        compiler_params=pltpu.CompilerParams(use_tc_tiling_on_sc=False),
        out_type=jax.ShapeDtypeStruct((_E, 4), jnp.float32),
        scratch_types=[
            pltpu.VMEM((_CH,), jnp.int32),
            pltpu.VMEM((_CH, 4), jnp.float32),
            pltpu.SemaphoreType.DMA,
        ],
    )
    def k(idx_hbm, ptbl_hbm, out_hbm, idx_v, rows_v, sem):
        wid = lax.axis_index("s") * _NC + lax.axis_index("c")
        base = wid * _EPW
        for ci in range(_EPW // _CH):
            b = base + ci * _CH
            pltpu.sync_copy(idx_hbm.at[pl.ds(b, _CH)], idx_v)
            pltpu.async_copy(ptbl_hbm.at[idx_v], rows_v, sem).wait()
            pltpu.sync_copy(rows_v, out_hbm.at[pl.ds(b, _CH)])

    return k(idx, ptbl)


_BA = 1000


def _tc_body(rows_ref, p_ref, s_ref, q_ref, o_ref):
    R = rows_ref[...]
    S = s_ref[...]
    xyzt = lax.dot_general(R, S, (((1,), (0,)), ((), ())),
                           precision=lax.Precision.HIGHEST)
    X, Y, Z, TJ = (xyzt[:, 0:16], xyzt[:, 16:32],
                   xyzt[:, 32:48], xyzt[:, 48:64])
    P = p_ref[...]
    dx = X - P[:, 0:1]
    dy = Y - P[:, 1:2]
    dz = Z - P[:, 2:3]
    ti = P[:, 3:4]
    r = jnp.sqrt(dx * dx + dy * dy + dz * dz)
    u = r * (1.0 / _R_C)
    hfc = jnp.where(r < _R_C, 0.25 * jnp.cos(jnp.pi * u) + 0.25, 0.0)
    x = 2.0 * (u - 1.0) * (u - 1.0) - 1.0
    t1 = x
    t2 = 2.0 * x * x - 1.0
    t3 = 2.0 * x * t2 - t1
    fns = [2.0 * hfc, (t1 + 1.0) * hfc, (t2 + 1.0) * hfc, (t3 + 1.0) * hfc]
    cols = []
    for tp in range(_N_TYPES):
        mask = (TJ == float(tp)).astype(jnp.float32)
        for kk in range(_K_MAX):
            cols.append(mask * fns[kk])
    W = jnp.concatenate(cols, axis=1)
    G4 = lax.dot_general(W, q_ref[...], (((1,), (0,)), ((), ())),
                         precision=lax.Precision.HIGHEST)
    acc = jnp.zeros((_BA, _N_DESC), jnp.float32)
    for t in range(_N_TYPES):
        m = (ti == float(t)).astype(jnp.float32)
        acc = acc + m * G4[:, _N_DESC * t:_N_DESC * (t + 1)]
    o_ref[...] = acc


def _sel_const():
    s = np.zeros((64, 64), np.float32)
    for j in range(_NN):
        for c in range(4):
            s[4 * j + c, c * 16 + j] = 1.0
    return s


def _tc_math(rows_a, ptbl, sel, q):
    return pl.pallas_call(
        _tc_body,
        grid=(_N_ATOMS // _BA,),
        in_specs=[
            pl.BlockSpec((_BA, 64), lambda i: (i, 0)),
            pl.BlockSpec((_BA, 4), lambda i: (i, 0)),
            pl.BlockSpec((64, 64), lambda i: (0, 0)),
            pl.BlockSpec((256, 64), lambda i: (0, 0)),
        ],
        out_specs=pl.BlockSpec((_BA, _N_DESC), lambda i: (i, 0)),
        out_shape=jax.ShapeDtypeStruct((_N_ATOMS, _N_DESC), jnp.float32),
    )(rows_a, ptbl, sel, q)


def kernel(types, positions, radial_neighbors, c_table):
    ptbl = jnp.concatenate(
        [positions.astype(jnp.float32), types.astype(jnp.float32)[:, None]],
        axis=1)
    idx = radial_neighbors.astype(jnp.int32).reshape(-1)
    rows = _sc_gather(idx, ptbl)
    rows_a = rows.reshape(_N_ATOMS, 64)
    qc = jnp.transpose(c_table.astype(jnp.float32), (1, 3, 0, 2))
    q = jnp.broadcast_to(qc.reshape(16, 1, 64), (16, 16, 64)).reshape(256, 64)
    sel = jnp.asarray(_sel_const())
    return _tc_math(rows_a, ptbl, sel, q)


# speedup vs baseline: 167.4623x; 9.0252x over previous
import functools

import numpy as np
import jax
import jax.numpy as jnp
from jax import lax
from jax.experimental import pallas as pl
from jax.experimental.pallas import tpu as pltpu
from jax.experimental.pallas import tpu_sc as plsc

_N_ATOMS = 100000
_NN = 16
_N_TYPES = 4
_N_DESC = 16
_K_MAX = 4
_R_C = 5.0

_NP = 102400
_NC = 2
_NS = 16
_NW = _NC * _NS
_APW = _NP // _NW
_CA = 640
_NCH = _APW // _CA


def _sc_gather_t(idxt, tabs):
    mesh = plsc.VectorSubcoreMesh(core_axis_name="c", subcore_axis_name="s")

    @functools.partial(
        pl.kernel,
        mesh=mesh,
        compiler_params=pltpu.CompilerParams(use_tc_tiling_on_sc=False,
                                             needs_layout_passes=False),
        out_type=jax.ShapeDtypeStruct((3, _NN, _NP), jnp.float32),
        scratch_types=[
            pltpu.VMEM((_N_ATOMS,), jnp.float32),
            pltpu.VMEM((_NN, _CA), jnp.int32),
            pltpu.VMEM((_NN, _CA), jnp.float32),
        ],
    )
    def k(idxt_hbm, tabs_hbm, out_hbm, tab_v, idxt_v, out_v):
        wid = lax.axis_index("s") * _NC + lax.axis_index("c")
        for c in range(3):
            pltpu.sync_copy(tabs_hbm.at[c], tab_v)
            for ci in range(_NCH):
                col0 = wid * _APW + ci * _CA
                pltpu.sync_copy(idxt_hbm.at[:, pl.ds(col0, _CA)], idxt_v)

                @plsc.parallel_loop(0, _CA // 16, unroll=2)
                def body(g):
                    a0 = g * 16
                    nbrs = [idxt_v[j, pl.ds(a0, 16)] for j in range(_NN)]
                    vals = [plsc.load_gather(tab_v, [nbrs[j]])
                            for j in range(_NN)]
                    for j in range(_NN):
                        out_v[j, pl.ds(a0, 16)] = vals[j]

                pltpu.sync_copy(out_v, out_hbm.at[c, :, pl.ds(col0, _CA)])

    return k(idxt, tabs)


_BAT = 2048


def _tc_body3(cmp_ref, pos_ref, ti_ref, qt_ref, o_ref):
    c3 = cmp_ref[...]
    xj, yj, zj = c3[0], c3[1], c3[2]
    tj = lax.bitcast_convert_type(xj, jnp.int32) & 3
    pp = pos_ref[...]
    dx = xj - pp[0:1, :]
    dy = yj - pp[1:2, :]
    dz = zj - pp[2:3, :]
    ti = ti_ref[...]
    r2 = dx * dx + dy * dy + dz * dz
    r = jnp.sqrt(r2)
    u = r * (1.0 / _R_C)
    w = 2.0 * u - 1.0
    w2 = w * w
    s = w * (1.570792378137 + w2 * (-0.645905999200 + w2 *
             (0.079464822790 + w2 * -0.004352781890)))
    hfc = jnp.where(r < _R_C, 0.25 - 0.25 * s, 0.0)
    x = 2.0 * (u - 1.0) * (u - 1.0) - 1.0
    t2 = 2.0 * x * x - 1.0
    t3 = 2.0 * x * t2 - x
    fns = [2.0 * hfc, (x + 1.0) * hfc, (t2 + 1.0) * hfc, (t3 + 1.0) * hfc]
    rows = []
    for tp in range(_N_TYPES):
        m = (tj == tp)
        for kk in range(_K_MAX):
            rows.append(jnp.where(m, fns[kk], 0.0))
    phi = jnp.concatenate(rows, axis=0).astype(jnp.bfloat16)
    g4 = lax.dot_general(qt_ref[...], phi, (((1,), (0,)), ((), ())),
                         preferred_element_type=jnp.float32)
    acc = jnp.zeros((_NN, _BAT), jnp.float32)
    for t in range(_N_TYPES):
        m = (ti == t)
        acc = acc + jnp.where(m, g4[_N_DESC * t:_N_DESC * (t + 1), :], 0.0)
    o_ref[...] = acc.T


def _tc_math3(cmps, post, tii, qt):
    return pl.pallas_call(
        _tc_body3,
        grid=(_NP // _BAT,),
        in_specs=[
            pl.BlockSpec((3, _NN, _BAT), lambda i: (0, 0, i)),
            pl.BlockSpec((3, _BAT), lambda i: (0, i)),
            pl.BlockSpec((1, _BAT), lambda i: (0, i)),
            pl.BlockSpec((64, 256), lambda i: (0, 0)),
        ],
        out_specs=pl.BlockSpec((_BAT, _NN), lambda i: (i, 0)),
        out_shape=jax.ShapeDtypeStruct((_NP, _NN), jnp.float32),
    )(cmps, post, tii, qt)


def _qt_const(c_table):
    base = jnp.transpose(c_table.astype(jnp.float32), (0, 2, 1, 3))
    qt16 = base.reshape(64, 16)
    qt = jnp.broadcast_to(qt16[:, :, None], (64, 16, 16)).reshape(64, 256)
    return qt.astype(jnp.bfloat16)


def kernel(types, positions, radial_neighbors, c_table):
    pos = positions.astype(jnp.float32)
    ti32 = types.astype(jnp.int32)
    xbits = lax.bitcast_convert_type(pos[:, 0], jnp.int32)
    xenc = lax.bitcast_convert_type((xbits & ~jnp.int32(3)) | ti32,
                                    jnp.float32)
    tabs = jnp.stack([xenc, pos[:, 1], pos[:, 2]], axis=0)
    idxt = jnp.pad(radial_neighbors.astype(jnp.int32),
                   ((0, _NP - _N_ATOMS), (0, 0))).T
    cmps = _sc_gather_t(idxt, tabs)
    post = jnp.pad(pos, ((0, _NP - _N_ATOMS), (0, 0))).T
    tii = jnp.pad(ti32, (0, _NP - _N_ATOMS)).reshape(1, _NP)
    return _tc_math3(cmps, post, tii, _qt_const(c_table))[:_N_ATOMS]
